# baseline (speedup 1.0000x reference)
import functools

import jax
import jax.numpy as jnp
from jax import lax
from jax.experimental import pallas as pl
from jax.experimental.pallas import tpu as pltpu
from jax.experimental.pallas import tpu_sc as plsc

NC = 2
NS = 16
L = 16

B = 4096
H = 200
E = 64
CTROWS = 11520

HT = H // 8
BT = B // 128
ET = E // 8
HG = 4
H_PER_G = H // HG
H_PER_SC = H // NC


def _ctt_kernel(ytt_ref, mte_ref, dte_ref, ctt_ref):
    ytt = ytt_ref[...]
    mte = mte_ref[...]
    dte = dte_ref[...]
    s = ytt[:, :, None] + (mte + dte)[:, None, :]
    ctt_ref[...] = jnp.maximum(s, 0.0)


mesh = plsc.VectorSubcoreMesh(core_axis_name="c", subcore_axis_name="s")


@functools.partial(
    pl.kernel,
    out_type=jax.ShapeDtypeStruct((H, ET, BT, 8, 128), jnp.float32),
    mesh=mesh,
    scratch_types=[
        pltpu.VMEM((4 * CTROWS,), jnp.float32),
        pltpu.VMEM((BT, 128), jnp.int32),
        pltpu.VMEM((BT, 128), jnp.int32),
        pltpu.VMEM((16, 4, 128), jnp.float32),
        pltpu.VMEM((16, 4, 128), jnp.float32),
        pltpu.VMEM((16, 4, 128), jnp.float32),
        pltpu.VMEM((16, 4, 128), jnp.float32),
        pltpu.VMEM((BT, 128), jnp.int32),
        pltpu.VMEM_SHARED((H_PER_SC, BT, 128), jnp.int32),
        pltpu.SemaphoreType.DMA,
        pltpu.SemaphoreType.DMA,
        pltpu.SemaphoreType.DMA,
        pltpu.SemaphoreType.DMA,
        pltpu.SemaphoreType.DMA,
        pltpu.SemaphoreType.DMA,
    ],
    compiler_params=pltpu.CompilerParams(
        use_tc_tiling_on_sc=False, needs_layout_passes=False),
)
def _sc_kernel(ctt_hbm, y4_hbm, m4_hbm, d4_hbm, out_hbm,
               ctt_v, c_v0, c_v1, slab_v0, slab_v1, slab_v2, slab_v3,
               t_r, c_sh, csem0, csem1,
               osem0, osem1, osem2, osem3):
    sc = lax.axis_index("c")
    tid = lax.axis_index("s")
    et = tid // 2
    ei0 = (tid % 2) * 4

    for j in range(4):
        pltpu.sync_copy(ctt_hbm.at[tid * 4 + j],
                        ctt_v.at[pl.ds(j * CTROWS, CTROWS)])

    rows_per_tile = (H_PER_SC + NS - 1) // NS

    def phase_a(k, carry):
        l = tid * rows_per_tile + k

        @pl.when(l < H_PER_SC)
        def _():
            h = sc * H_PER_SC + l
            ht = h // 8
            hi = h % 8
            pltpu.sync_copy(y4_hbm.at[ht, :, hi, :], t_r)
            for u in range(BT):
                for v in range(8):
                    sl = pl.ds(v * L, L)
                    c_v0[u, sl] = t_r[u, sl] * 384
            pltpu.sync_copy(m4_hbm.at[ht, :, hi, :], t_r)
            for u in range(BT):
                for v in range(8):
                    sl = pl.ds(v * L, L)
                    c_v0[u, sl] = c_v0[u, sl] + t_r[u, sl] * 32
            pltpu.sync_copy(d4_hbm.at[ht, :, hi, :], t_r)
            for u in range(BT):
                for v in range(8):
                    sl = pl.ds(v * L, L)
                    c_v0[u, sl] = c_v0[u, sl] + t_r[u, sl]
            pltpu.sync_copy(c_v0, c_sh.at[l])

        return carry

    lax.fori_loop(0, rows_per_tile, phase_a, 0)
    plsc.subcore_barrier()

    def gather_slab(c_v, slab_v, bt0):
        @plsc.parallel_loop(0, 16, step=1, unroll=1)
        def _(t):
            bt = bt0 + t
            cw = [c_v[bt, pl.ds(bl * L, L)] for bl in range(8)]
            for ei in range(4):
                base = ei * CTROWS
                for bl in range(8):
                    g = plsc.load_gather(ctt_v, [cw[bl] + base])
                    slab_v[t, ei, pl.ds(bl * L, L)] = g

    def c_in(l, c_v, csem):
        return pltpu.make_async_copy(c_sh.at[l], c_v, csem)

    def slab_out(h, slab_v, bt0, osem):
        return pltpu.make_async_copy(
            slab_v, out_hbm.at[h, et, pl.ds(bt0, 16), pl.ds(ei0, 4), :], osem)

    h0 = sc * H_PER_SC
    c_in(0, c_v0, csem0).start()

    def do_pair(k, carry):
        l = 2 * k

        def do_h(h, c_v, slab_a, sem_a, slab_b, sem_b):
            @pl.when(k > 0)
            def _():
                slab_out(h, slab_a, 0, sem_a).wait()

            gather_slab(c_v, slab_a, 0)
            slab_out(h, slab_a, 0, sem_a).start()

            @pl.when(k > 0)
            def _():
                slab_out(h, slab_b, 16, sem_b).wait()

            gather_slab(c_v, slab_b, 16)
            slab_out(h, slab_b, 16, sem_b).start()

        c_in(l, c_v0, csem0).wait()
        c_in(l + 1, c_v1, csem1).start()
        do_h(h0 + l, c_v0, slab_v0, osem0, slab_v1, osem1)

        c_in(l + 1, c_v1, csem1).wait()

        @pl.when(k + 1 < H_PER_SC // 2)
        def _():
            c_in(l + 2, c_v0, csem0).start()

        do_h(h0 + l + 1, c_v1, slab_v2, osem2, slab_v3, osem3)
        return carry

    lax.fori_loop(0, H_PER_SC // 2, do_pair, 0)
    slab_out(h0, slab_v0, 0, osem0).wait()
    slab_out(h0, slab_v1, 16, osem1).wait()
    slab_out(h0, slab_v2, 0, osem2).wait()
    slab_out(h0, slab_v3, 16, osem3).wait()


def kernel(years, months, days, year_table, month_table, day_table):
    f32 = jnp.float32

    ytt = year_table.T
    mtt = month_table.T
    dtt = jnp.zeros((E, 32), f32).at[:, :31].set(day_table.T)
    mte = jnp.repeat(mtt, 32, axis=1)
    dte = jnp.tile(dtt, (1, 12))

    ct4 = pl.pallas_call(
        _ctt_kernel,
        out_shape=jax.ShapeDtypeStruct((E, 30, 384), f32),
    )(ytt, mte, dte)
    ctt = ct4.reshape(E, CTROWS)

    def tiles(a):
        return a.astype(jnp.int32).T.reshape(HT, 8, BT, 128).transpose(0, 2, 1, 3)

    o = _sc_kernel(ctt, tiles(years), tiles(months), tiles(days))
    return o.transpose(2, 4, 0, 1, 3).reshape(B, H, E)


# speedup vs baseline: 1.0805x; 1.0805x over previous
import functools

import jax
import jax.numpy as jnp
from jax import lax
from jax.experimental import pallas as pl
from jax.experimental.pallas import tpu as pltpu
from jax.experimental.pallas import tpu_sc as plsc

NC = 2
NS = 16
L = 16

B = 4096
H = 200
E = 64
CTROWS = 11520

HT = H // 8
BT = B // 128
ET = E // 8
HG = 4
H_PER_G = H // HG
H_PER_SC = H // NC


def _ctt_kernel(ytt_ref, mte_ref, dte_ref, ctt_ref):
    ytt = ytt_ref[...]
    mte = mte_ref[...]
    dte = dte_ref[...]
    s = ytt[:, :, None] + (mte + dte)[:, None, :]
    ctt_ref[...] = jnp.maximum(s, 0.0)


mesh = plsc.VectorSubcoreMesh(core_axis_name="c", subcore_axis_name="s")


@functools.partial(
    pl.kernel,
    out_type=jax.ShapeDtypeStruct((H, ET, BT, 8, 128), jnp.float32),
    mesh=mesh,
    scratch_types=[
        pltpu.VMEM((4 * CTROWS,), jnp.float32),
        pltpu.VMEM((BT, 128), jnp.int32),
        pltpu.VMEM((BT, 128), jnp.int32),
        pltpu.VMEM((16, 4, 128), jnp.float32),
        pltpu.VMEM((16, 4, 128), jnp.float32),
        pltpu.VMEM((BT, 128), jnp.int32),
        pltpu.VMEM((BT, 128), jnp.int32),
        pltpu.VMEM((BT, 128), jnp.int32),
        pltpu.VMEM_SHARED((H_PER_SC, BT, 128), jnp.int32),
        pltpu.SemaphoreType.DMA,
        pltpu.SemaphoreType.DMA,
        pltpu.SemaphoreType.DMA,
        pltpu.SemaphoreType.DMA,
        pltpu.SemaphoreType.DMA,
    ],
    compiler_params=pltpu.CompilerParams(
        use_tc_tiling_on_sc=False, needs_layout_passes=False),
)
def _sc_kernel(ctt_hbm, y4_hbm, m4_hbm, d4_hbm, out_hbm,
               ctt_v, c_v0, c_v1, slab_v0, slab_v1, y_r, m_r, d_r, c_sh,
               csem0, csem1, osem0, osem1, tsem):
    sc = lax.axis_index("c")
    tid = lax.axis_index("s")
    et = tid // 2
    ei0 = (tid % 2) * 4

    ctt_copies = [
        pltpu.make_async_copy(ctt_hbm.at[tid * 4 + j],
                              ctt_v.at[pl.ds(j * CTROWS, CTROWS)], tsem)
        for j in range(4)
    ]
    for cp in ctt_copies:
        cp.start()

    rows_per_tile = (H_PER_SC + NS - 1) // NS

    def phase_a(k, carry):
        l = tid * rows_per_tile + k

        @pl.when(l < H_PER_SC)
        def _():
            h = sc * H_PER_SC + l
            ht = h // 8
            hi = h % 8
            pltpu.sync_copy(y4_hbm.at[ht, :, hi, :], y_r)
            pltpu.sync_copy(m4_hbm.at[ht, :, hi, :], m_r)
            pltpu.sync_copy(d4_hbm.at[ht, :, hi, :], d_r)
            for u in range(BT):
                for v in range(8):
                    sl = pl.ds(v * L, L)
                    c = (y_r[u, sl] * 12 + m_r[u, sl]) * 32 + d_r[u, sl]
                    c_v0[u, sl] = c
            pltpu.sync_copy(c_v0, c_sh.at[l])

        return carry

    lax.fori_loop(0, rows_per_tile, phase_a, 0)
    for cp in ctt_copies:
        cp.wait()
    plsc.subcore_barrier()

    def gather_slab(c_v, slab_v, bt0):
        @plsc.parallel_loop(0, 16, step=1, unroll=1)
        def _(t):
            bt = bt0 + t
            cw = [c_v[bt, pl.ds(bl * L, L)] for bl in range(8)]
            for ei in range(4):
                base = ei * CTROWS
                for bl in range(8):
                    g = plsc.load_gather(ctt_v, [cw[bl] + base])
                    slab_v[t, ei, pl.ds(bl * L, L)] = g

    def c_in(l, c_v, csem):
        return pltpu.make_async_copy(c_sh.at[l], c_v, csem)

    def slab_out(h, slab_v, bt0, osem):
        return pltpu.make_async_copy(
            slab_v, out_hbm.at[h, et, pl.ds(bt0, 16), pl.ds(ei0, 4), :], osem)

    h0 = sc * H_PER_SC
    c_in(0, c_v0, csem0).start()

    def do_pair(k, carry):
        l = 2 * k

        def do_h(h, c_v, slab_a, slab_b):
            @pl.when(h > h0)
            def _():
                slab_out(h, slab_a, 0, osem0).wait()

            gather_slab(c_v, slab_a, 0)
            slab_out(h, slab_a, 0, osem0).start()

            @pl.when(h > h0)
            def _():
                slab_out(h, slab_b, 16, osem1).wait()

            gather_slab(c_v, slab_b, 16)
            slab_out(h, slab_b, 16, osem1).start()

        c_in(l, c_v0, csem0).wait()
        c_in(l + 1, c_v1, csem1).start()
        do_h(h0 + l, c_v0, slab_v0, slab_v1)

        c_in(l + 1, c_v1, csem1).wait()

        @pl.when(k + 1 < H_PER_SC // 2)
        def _():
            c_in(l + 2, c_v0, csem0).start()

        do_h(h0 + l + 1, c_v1, slab_v0, slab_v1)
        return carry

    lax.fori_loop(0, H_PER_SC // 2, do_pair, 0)
    slab_out(h0, slab_v0, 0, osem0).wait()
    slab_out(h0, slab_v1, 16, osem1).wait()


def kernel(years, months, days, year_table, month_table, day_table):
    f32 = jnp.float32

    ytt = year_table.T
    mtt = month_table.T
    dtt = jnp.zeros((E, 32), f32).at[:, :31].set(day_table.T)
    mte = jnp.repeat(mtt, 32, axis=1)
    dte = jnp.tile(dtt, (1, 12))

    ct4 = pl.pallas_call(
        _ctt_kernel,
        out_shape=jax.ShapeDtypeStruct((E, 30, 384), f32),
    )(ytt, mte, dte)
    ctt = ct4.reshape(E, CTROWS)

    def tiles(a):
        return a.astype(jnp.int32).T.reshape(HT, 8, BT, 128).transpose(0, 2, 1, 3)

    o = _sc_kernel(ctt, tiles(years), tiles(months), tiles(days))
    return o.transpose(2, 4, 0, 1, 3).reshape(B, H, E)
